# Initial kernel scaffold; baseline (speedup 1.0000x reference)
import jax
import jax.numpy as jnp
from jax.experimental import pallas as pl


def kernel(x, table):
    raise NotImplementedError("write your pallas kernel here")


# speedup vs baseline: 1.0948x; 1.0948x over previous
import functools

import jax
import jax.numpy as jnp
from jax import lax
from jax.experimental import pallas as pl
from jax.experimental.pallas import tpu as pltpu
from jax.experimental.pallas import tpu_sc as plsc


def _make_gather(V, D, B):
    info = plsc.get_sparse_core_info()
    NC, NS = info.num_cores, info.num_subcores
    NW = NC * NS
    assert B % NW == 0
    b_per_w = B // NW
    C = 1024
    while b_per_w % C:
        C //= 2
    n_chunks = b_per_w // C

    mesh = plsc.VectorSubcoreMesh(core_axis_name="c", subcore_axis_name="s")

    @functools.partial(
        pl.kernel,
        mesh=mesh,
        compiler_params=pltpu.CompilerParams(use_tc_tiling_on_sc=False),
        out_type=jax.ShapeDtypeStruct((B, D), jnp.float32),
        scratch_types=[
            pltpu.VMEM((C,), jnp.int32),
            pltpu.VMEM((C, D), jnp.float32),
            pltpu.SemaphoreType.DMA,
        ],
    )
    def gather_kernel(table_hbm, idx_hbm, out_hbm, idx_v, rows_v, sem):
        wid = lax.axis_index("s") * NC + lax.axis_index("c")
        base = wid * b_per_w

        def body(i, carry):
            off = pl.multiple_of(base + i * C, C)
            pltpu.sync_copy(idx_hbm.at[pl.ds(off, C)], idx_v)
            pltpu.async_copy(table_hbm.at[idx_v], rows_v, sem).wait()
            pltpu.sync_copy(rows_v, out_hbm.at[pl.ds(off, C)])
            return carry

        lax.fori_loop(0, n_chunks, body, 0)

    return gather_kernel


def kernel(x, table):
    V, D = table.shape
    B = x.size
    idx = x.reshape(B).astype(jnp.int32)
    out = _make_gather(V, D, B)(table, idx)
    return out.reshape(x.shape + (D,))


# speedup vs baseline: 1.1133x; 1.0169x over previous
import functools

import jax
import jax.numpy as jnp
from jax import lax
from jax.experimental import pallas as pl
from jax.experimental.pallas import tpu as pltpu
from jax.experimental.pallas import tpu_sc as plsc

_R = 4


def _make_gather(V, D, B, C):
    info = plsc.get_sparse_core_info()
    NC, NS = info.num_cores, info.num_subcores
    NW = NC * NS
    assert B % NW == 0
    b_per_w = B // NW
    assert b_per_w % C == 0 and C % 8 == 0
    n = b_per_w // C
    assert n % _R == 0 and n >= 2 * _R
    G = n // _R

    mesh = plsc.VectorSubcoreMesh(core_axis_name="c", subcore_axis_name="s")

    @functools.partial(
        pl.kernel,
        mesh=mesh,
        compiler_params=pltpu.CompilerParams(use_tc_tiling_on_sc=False),
        out_type=jax.ShapeDtypeStruct((B, D), jnp.float32),
        scratch_types=[
            pltpu.VMEM((_R, C), jnp.int32),
            pltpu.VMEM((_R, C, D), jnp.float32),
            pltpu.SemaphoreType.DMA((_R,)),
            pltpu.SemaphoreType.DMA((_R,)),
            pltpu.SemaphoreType.DMA((_R,)),
        ],
    )
    def gather_kernel(table_hbm, idx_hbm, out_hbm, idx_v, rows_v, sem_i, sem_g, sem_o):
        wid = lax.axis_index("s") * NC + lax.axis_index("c")
        base = wid * b_per_w

        def idx_copy(k, j):
            off = pl.multiple_of(base + (k % n) * C, 8)
            return pltpu.make_async_copy(
                idx_hbm.at[pl.ds(off, C)], idx_v.at[j], sem_i.at[j])

        def gather_copy(j):
            return pltpu.make_async_copy(
                table_hbm.at[idx_v.at[j]], rows_v.at[j], sem_g.at[j])

        def out_copy(k, j):
            off = pl.multiple_of(base + k * C, 8)
            return pltpu.make_async_copy(
                rows_v.at[j], out_hbm.at[pl.ds(off, C)], sem_o.at[j])

        for j in range(_R):
            idx_copy(j, j).start()
        for j in range(_R):
            idx_copy(j, j).wait()
            gather_copy(j).start()
            if j > 0:
                p = j - 1
                gather_copy(p).wait()
                out_copy(p, p).start()
                idx_copy(p + _R, p).start()

        def body(g, carry):
            for j in range(_R):
                i = g * _R + j
                p = (j + _R - 1) % _R
                idx_copy(i, j).wait()
                out_copy(i - _R, j).wait()
                gather_copy(j).start()
                gather_copy(p).wait()
                out_copy(i - 1, p).start()
                idx_copy(i - 1 + _R, p).start()
            return carry

        lax.fori_loop(1, G, body, 0)

        p = (n - 1) % _R
        gather_copy(p).wait()
        out_copy(n - 1, p).start()
        for j in range(_R):
            out_copy(0, j).wait()
        for j in range(_R - 1):
            idx_copy(0, j).wait()

    return gather_kernel


def kernel(x, table):
    V, D = table.shape
    B = x.size
    idx = x.reshape(B).astype(jnp.int32)
    out = _make_gather(V, D, B, 800)(table, idx)
    return out.reshape(x.shape + (D,))
